# Initial kernel scaffold; baseline (speedup 1.0000x reference)
import jax
import jax.numpy as jnp
from jax.experimental import pallas as pl


def kernel(x, W):
    raise NotImplementedError("write your pallas kernel here")


# speedup vs baseline: 9.0901x; 9.0901x over previous
import functools

import jax
import jax.numpy as jnp
from jax import lax
from jax.experimental import pallas as pl
from jax.experimental.pallas import tpu as pltpu
from jax.experimental.pallas import tpu_sc as plsc

_VOCAB = 1000
_DIM = 128
_FEATURES = 26
_BATCH = 16384

_NUM_CORES = 2
_NUM_SUBCORES = 16
_NUM_WORKERS = _NUM_CORES * _NUM_SUBCORES
_ROWS_PER_WORKER = _BATCH // _NUM_WORKERS
_CHUNK = 128
_NUM_CHUNKS = _ROWS_PER_WORKER // _CHUNK
_LANES = 16

_mesh = plsc.VectorSubcoreMesh(core_axis_name="c", subcore_axis_name="s")


@functools.partial(
    pl.kernel,
    out_type=jax.ShapeDtypeStruct((_BATCH, _DIM), jnp.float32),
    mesh=_mesh,
    scratch_types=[
        pltpu.VMEM((_FEATURES, _NUM_CHUNKS, _CHUNK), jnp.int32),
        pltpu.VMEM((_CHUNK, _DIM), jnp.float32),
        pltpu.VMEM((_CHUNK, _DIM), jnp.float32),
        pltpu.VMEM((_CHUNK, _DIM), jnp.float32),
        pltpu.SemaphoreType.DMA,
        pltpu.SemaphoreType.DMA,
        pltpu.SemaphoreType.DMA,
    ],
)
def _embed_sum(w_hbm, idx_hbm, out_hbm, idx_v, acc, st0, st1,
               sem_a, sem0, sem1):
    wid = lax.axis_index("s") * _NUM_CORES + lax.axis_index("c")
    base = wid * _ROWS_PER_WORKER
    pltpu.sync_copy(idx_hbm.at[:, wid], idx_v)

    def accumulate(st):
        def body(r, carry):
            for c in range(_DIM // _LANES):
                sl = pl.ds(c * _LANES, _LANES)
                plsc.addupdate(acc.at[r, sl], st[r, sl])
            return carry
        lax.fori_loop(0, _CHUNK, body, 0, unroll=2)

    for ch in range(_NUM_CHUNKS):
        c_acc = pltpu.async_copy(w_hbm.at[idx_v.at[0, ch]], acc, sem_a)
        pending = pltpu.async_copy(w_hbm.at[idx_v.at[1, ch]], st0, sem0)
        c_acc.wait()
        for i in range(1, _FEATURES):
            cur = st0 if i % 2 == 1 else st1
            nxt = None
            if i + 1 < _FEATURES:
                nxt_buf = st1 if i % 2 == 1 else st0
                nxt_sem = sem1 if i % 2 == 1 else sem0
                nxt = pltpu.async_copy(
                    w_hbm.at[idx_v.at[i + 1, ch]], nxt_buf, nxt_sem)
            pending.wait()
            accumulate(cur)
            pending = nxt
        pltpu.sync_copy(acc, out_hbm.at[pl.ds(base + ch * _CHUNK, _CHUNK)])


@jax.jit
def kernel(x, W):
    offs = jnp.arange(_FEATURES, dtype=jnp.int32) * _VOCAB
    idx = (x.astype(jnp.int32) + offs[None, :]).T.reshape(
        _FEATURES, _NUM_WORKERS, _NUM_CHUNKS, _CHUNK)
    w_flat = W.reshape(_FEATURES * _VOCAB, _DIM)
    return _embed_sum(w_flat, idx)


# speedup vs baseline: 9.3384x; 1.0273x over previous
import functools

import jax
import jax.numpy as jnp
from jax import lax
from jax.experimental import pallas as pl
from jax.experimental.pallas import tpu as pltpu
from jax.experimental.pallas import tpu_sc as plsc

_VOCAB = 1000
_DIM = 128
_FEATURES = 26
_BATCH = 16384

_NUM_CORES = 2
_NUM_SUBCORES = 16
_NUM_WORKERS = _NUM_CORES * _NUM_SUBCORES
_ROWS_PER_WORKER = _BATCH // _NUM_WORKERS
_CHUNK = 128
_NUM_CHUNKS = _ROWS_PER_WORKER // _CHUNK
_LANES = 16

_mesh = plsc.VectorSubcoreMesh(core_axis_name="c", subcore_axis_name="s")


@functools.partial(
    pl.kernel,
    out_type=jax.ShapeDtypeStruct((_BATCH, _DIM), jnp.float32),
    mesh=_mesh,
    scratch_types=[
        pltpu.VMEM((_FEATURES, _NUM_CHUNKS, _CHUNK), jnp.int32),
        pltpu.VMEM((_CHUNK, _DIM), jnp.float32),
        pltpu.VMEM((_CHUNK, _DIM), jnp.float32),
        pltpu.VMEM((_CHUNK, _DIM), jnp.float32),
        pltpu.SemaphoreType.DMA,
        pltpu.SemaphoreType.DMA,
        pltpu.SemaphoreType.DMA,
    ],
)
def _embed_sum(w_hbm, idx_hbm, out_hbm, idx_v, acc, st0, st1,
               sem_a, sem0, sem1):
    wid = lax.axis_index("s") * _NUM_CORES + lax.axis_index("c")
    base = wid * _ROWS_PER_WORKER
    pltpu.sync_copy(idx_hbm.at[:, wid], idx_v)

    def accumulate(st):
        @plsc.parallel_loop(0, _CHUNK, 1, unroll=4)
        def _(r):
            for c in range(_DIM // _LANES):
                sl = pl.ds(c * _LANES, _LANES)
                plsc.addupdate(acc.at[r, sl], st[r, sl])

    def chunk_body(ch, carry):
        c_acc = pltpu.async_copy(w_hbm.at[idx_v.at[0, ch]], acc, sem_a)
        pending = pltpu.async_copy(w_hbm.at[idx_v.at[1, ch]], st0, sem0)
        c_acc.wait()
        for i in range(1, _FEATURES):
            cur = st0 if i % 2 == 1 else st1
            nxt = None
            if i + 1 < _FEATURES:
                nxt_buf = st1 if i % 2 == 1 else st0
                nxt_sem = sem1 if i % 2 == 1 else sem0
                nxt = pltpu.async_copy(
                    w_hbm.at[idx_v.at[i + 1, ch]], nxt_buf, nxt_sem)
            pending.wait()
            accumulate(cur)
            pending = nxt
        pltpu.sync_copy(acc, out_hbm.at[pl.ds(base + ch * _CHUNK, _CHUNK)])
        return carry

    lax.fori_loop(0, _NUM_CHUNKS, chunk_body, 0)


@jax.jit
def kernel(x, W):
    offs = jnp.arange(_FEATURES, dtype=jnp.int32) * _VOCAB
    idx = (x.astype(jnp.int32) + offs[None, :]).T.reshape(
        _FEATURES, _NUM_WORKERS, _NUM_CHUNKS, _CHUNK)
    w_flat = W.reshape(_FEATURES * _VOCAB, _DIM)
    return _embed_sum(w_flat, idx)


# speedup vs baseline: 10.0197x; 1.0730x over previous
import functools

import jax
import jax.numpy as jnp
from jax import lax
from jax.experimental import pallas as pl
from jax.experimental.pallas import tpu as pltpu
from jax.experimental.pallas import tpu_sc as plsc

_VOCAB = 1000
_DIM = 128
_FEATURES = 26
_BATCH = 16384

_NUM_CORES = 2
_NUM_SUBCORES = 16
_NUM_WORKERS = _NUM_CORES * _NUM_SUBCORES
_ROWS_PER_WORKER = _BATCH // _NUM_WORKERS
_CHUNK = 256
_NUM_CHUNKS = _ROWS_PER_WORKER // _CHUNK
_LANES = 16

_mesh = plsc.VectorSubcoreMesh(core_axis_name="c", subcore_axis_name="s")


@functools.partial(
    pl.kernel,
    out_type=jax.ShapeDtypeStruct((_BATCH, _DIM), jnp.float32),
    mesh=_mesh,
    scratch_types=[
        pltpu.VMEM((_FEATURES, _NUM_CHUNKS, _CHUNK), jnp.int32),
        pltpu.VMEM((_CHUNK, _DIM), jnp.float32),
        pltpu.VMEM((_CHUNK, _DIM), jnp.float32),
        pltpu.VMEM((_CHUNK, _DIM), jnp.float32),
        pltpu.SemaphoreType.DMA,
        pltpu.SemaphoreType.DMA,
        pltpu.SemaphoreType.DMA,
    ],
    compiler_params=pltpu.CompilerParams(use_tc_tiling_on_sc=False),
)
def _embed_sum(w_hbm, idx_hbm, out_hbm, idx_v, acc, st0, st1,
               sem_a, sem0, sem1):
    wid = lax.axis_index("s") * _NUM_CORES + lax.axis_index("c")
    base = wid * _ROWS_PER_WORKER
    pltpu.sync_copy(idx_hbm.at[:, wid], idx_v)

    def accumulate(st):
        @plsc.parallel_loop(0, _CHUNK, 1, unroll=4)
        def _(r):
            for c in range(_DIM // _LANES):
                sl = pl.ds(c * _LANES, _LANES)
                plsc.addupdate(acc.at[r, sl], st[r, sl])

    def chunk_body(ch, carry):
        c_acc = pltpu.async_copy(w_hbm.at[idx_v.at[0, ch]], acc, sem_a)
        pending = pltpu.async_copy(w_hbm.at[idx_v.at[1, ch]], st0, sem0)
        c_acc.wait()
        for i in range(1, _FEATURES):
            cur = st0 if i % 2 == 1 else st1
            nxt = None
            if i + 1 < _FEATURES:
                nxt_buf = st1 if i % 2 == 1 else st0
                nxt_sem = sem1 if i % 2 == 1 else sem0
                nxt = pltpu.async_copy(
                    w_hbm.at[idx_v.at[i + 1, ch]], nxt_buf, nxt_sem)
            pending.wait()
            accumulate(cur)
            pending = nxt
        pltpu.sync_copy(acc, out_hbm.at[pl.ds(base + ch * _CHUNK, _CHUNK)])
        return carry

    lax.fori_loop(0, _NUM_CHUNKS, chunk_body, 0)


@jax.jit
def kernel(x, W):
    offs = jnp.arange(_FEATURES, dtype=jnp.int32) * _VOCAB
    idx = (x.astype(jnp.int32) + offs[None, :]).T.reshape(
        _FEATURES, _NUM_WORKERS, _NUM_CHUNKS, _CHUNK)
    w_flat = W.reshape(_FEATURES * _VOCAB, _DIM)
    return _embed_sum(w_flat, idx)
